# Initial kernel scaffold; baseline (speedup 1.0000x reference)
import jax
import jax.numpy as jnp
from jax.experimental import pallas as pl


def kernel(feature_norm, label, kernel):
    raise NotImplementedError("write your pallas kernel here")


# speedup vs baseline: 8.0198x; 8.0198x over previous
import functools
import jax
import jax.numpy as jnp
from jax import lax
from jax.experimental import pallas as pl
from jax.experimental.pallas import tpu as pltpu, tpu_sc as plsc

_CLASSNUM = 70722
_EMBED = 512
_BATCH = 4096
_EPS = 0.001

_NC = 2
_NS = 16
_L = 16
_C_PAD = 70728
_CHUNK = _BATCH // _NS
_HALF = _CHUNK // 2
_OUT = _BATCH // (_NC * _NS)


def _newton_rsqrt(v):
    i = lax.bitcast_convert_type(v, jnp.int32)
    i = jnp.int32(0x5F3759DF) - lax.shift_right_logical(i, 1)
    y = lax.bitcast_convert_type(i, jnp.float32)
    for _ in range(3):
        y = y * (1.5 - 0.5 * v * y * y)
    return y


def _sc_body(label_hbm, fn_hbm, out_hbm,
             lab2, fnv2, sqv2, ones_v, zeros_v,
             cnt_g, sum_g, sq_g, res_v,
             counts_sh, sums_sh, sqs_sh):
    cid = lax.axis_index("c")
    sid = lax.axis_index("s")
    wid = sid * _NC + cid

    for j in range(2):
        base = sid * _CHUNK + j * _HALF
        pltpu.sync_copy(label_hbm.at[pl.ds(base, _HALF)], lab2.at[j])
        pltpu.sync_copy(fn_hbm.at[pl.ds(base, _HALF)], fnv2.at[j])

    for k in range(_HALF // _L):
        sl = pl.ds(k * _L, _L)
        ones_v[sl] = jnp.full((_L,), 1.0, jnp.float32)
        zeros_v[sl] = jnp.full((_L,), 0.0, jnp.float32)
        for j in range(2):
            f = fnv2[j, sl]
            sqv2[j, sl] = f * f

    for j in range(2):
        idx = lab2.at[j]
        pltpu.sync_copy(zeros_v, counts_sh.at[idx])
        pltpu.sync_copy(zeros_v, sums_sh.at[idx])
        pltpu.sync_copy(zeros_v, sqs_sh.at[idx])
    plsc.subcore_barrier()

    for j in range(2):
        idx = lab2.at[j]
        pltpu.sync_copy(ones_v, counts_sh.at[idx], add=True)
        pltpu.sync_copy(fnv2.at[j], sums_sh.at[idx], add=True)
        pltpu.sync_copy(sqv2.at[j], sqs_sh.at[idx], add=True)
    plsc.subcore_barrier()

    idx = lab2.at[cid]
    pltpu.sync_copy(counts_sh.at[idx], cnt_g)
    pltpu.sync_copy(sums_sh.at[idx], sum_g)
    pltpu.sync_copy(sqs_sh.at[idx], sq_g)

    for k in range(_OUT // _L):
        sl = pl.ds(k * _L, _L)
        cnt = cnt_g[sl]
        s = sum_g[sl]
        q = sq_g[sl]
        f = fnv2[cid, sl]
        mean = s / jnp.maximum(cnt, 1.0)
        var = (q - cnt * mean * mean) / jnp.maximum(cnt - 1.0, 1.0)
        var = jnp.maximum(var, 0.0)
        y = _newton_rsqrt(jnp.maximum(var, 1e-30))
        std = var * y
        d = f - mean
        res_v[sl] = jnp.where(cnt > 2.0, d / (std + _EPS), d / 20.0)

    pltpu.sync_copy(res_v, out_hbm.at[pl.ds(wid * _OUT, _OUT)])


@jax.jit
def _lamaface_sc(label, fn):
    mesh = plsc.VectorSubcoreMesh(core_axis_name="c", subcore_axis_name="s")
    run = pl.kernel(
        _sc_body,
        out_type=jax.ShapeDtypeStruct((_BATCH,), jnp.float32),
        mesh=mesh,
        scratch_types=[
            pltpu.VMEM((2, _HALF), jnp.int32),
            pltpu.VMEM((2, _HALF), jnp.float32),
            pltpu.VMEM((2, _HALF), jnp.float32),
            pltpu.VMEM((_HALF,), jnp.float32),
            pltpu.VMEM((_HALF,), jnp.float32),
            pltpu.VMEM((_OUT,), jnp.float32),
            pltpu.VMEM((_OUT,), jnp.float32),
            pltpu.VMEM((_OUT,), jnp.float32),
            pltpu.VMEM((_OUT,), jnp.float32),
            pltpu.VMEM_SHARED((_C_PAD,), jnp.float32),
            pltpu.VMEM_SHARED((_C_PAD,), jnp.float32),
            pltpu.VMEM_SHARED((_C_PAD,), jnp.float32),
        ],
    )
    return run(label, fn)


def kernel(feature_norm, label, kernel):
    del kernel
    res = _lamaface_sc(label, feature_norm[:, 0])
    return res[:, None]


# speedup vs baseline: 8.8055x; 1.0980x over previous
import jax
import jax.numpy as jnp
from jax import lax
from jax.experimental import pallas as pl
from jax.experimental.pallas import tpu as pltpu, tpu_sc as plsc

_CLASSNUM = 70722
_BATCH = 4096
_EPS = 0.001

_NC = 2
_NS = 16
_L = 16
_C_PAD = 70728
_CHUNK = _BATCH // _NS
_HALF = _CHUNK // 2
_OUT = _BATCH // (_NC * _NS)

_ONES, _ZEROS, _CNT, _SUM, _SQ, _RES = range(6)


def _newton_rsqrt(v):
    i = lax.bitcast_convert_type(v, jnp.int32)
    i = jnp.int32(0x5F3759DF) - lax.shift_right_logical(i, 1)
    y = lax.bitcast_convert_type(i, jnp.float32)
    for _ in range(3):
        y = y * (1.5 - 0.5 * v * y * y)
    return y


def _sc_body(label_hbm, fn_hbm, out_hbm,
             lab2, fnv2, sqv2, work,
             counts_sh, sums_sh, sqs_sh, sem_a, sem_b, sem_c):
    cid = lax.axis_index("c")
    sid = lax.axis_index("s")
    wid = sid * _NC + cid

    lab_d = []
    fn_d = []
    for j in range(2):
        base = sid * _CHUNK + j * _HALF
        lab_d.append(pltpu.async_copy(
            label_hbm.at[pl.ds(base, _HALF)], lab2.at[j], sem_a))
        fn_d.append(pltpu.async_copy(
            fn_hbm.at[pl.ds(base, _HALF)], fnv2.at[j], sem_b))

    for k in range(_HALF // _L):
        sl = pl.ds(k * _L, _L)
        work[_ONES, sl] = jnp.full((_L,), 1.0, jnp.float32)
        work[_ZEROS, sl] = jnp.full((_L,), 0.0, jnp.float32)

    for d in lab_d:
        d.wait()

    zero_d = []
    for j in range(2):
        idx = lab2.at[j]
        zero_d.append(pltpu.async_copy(work.at[_ZEROS], counts_sh.at[idx], sem_c))
        zero_d.append(pltpu.async_copy(work.at[_ZEROS], sums_sh.at[idx], sem_c))
        zero_d.append(pltpu.async_copy(work.at[_ZEROS], sqs_sh.at[idx], sem_c))

    for d in fn_d:
        d.wait()
    for k in range(_HALF // _L):
        sl = pl.ds(k * _L, _L)
        for j in range(2):
            f = fnv2[j, sl]
            sqv2[j, sl] = f * f

    for d in zero_d:
        d.wait()
    plsc.subcore_barrier()

    add_d = []
    for j in range(2):
        idx = lab2.at[j]
        add_d.append(pltpu.async_copy(
            work.at[_ONES], counts_sh.at[idx], sem_a, add=True))
        add_d.append(pltpu.async_copy(
            fnv2.at[j], sums_sh.at[idx], sem_a, add=True))
        add_d.append(pltpu.async_copy(
            sqv2.at[j], sqs_sh.at[idx], sem_a, add=True))
    for d in add_d:
        d.wait()
    plsc.subcore_barrier()

    idx = lab2.at[cid]
    gat_d = [
        pltpu.async_copy(counts_sh.at[idx], work.at[_CNT], sem_b),
        pltpu.async_copy(sums_sh.at[idx], work.at[_SUM], sem_b),
        pltpu.async_copy(sqs_sh.at[idx], work.at[_SQ], sem_b),
    ]
    for d in gat_d:
        d.wait()

    for k in range(_OUT // _L):
        sl = pl.ds(k * _L, _L)
        cnt = work[_CNT, sl]
        s = work[_SUM, sl]
        q = work[_SQ, sl]
        f = fnv2[cid, sl]
        mean = s / jnp.maximum(cnt, 1.0)
        var = (q - cnt * mean * mean) / jnp.maximum(cnt - 1.0, 1.0)
        var = jnp.maximum(var, 0.0)
        y = _newton_rsqrt(jnp.maximum(var, 1e-30))
        std = var * y
        d = f - mean
        work[_RES, sl] = jnp.where(cnt > 2.0, d / (std + _EPS), d / 20.0)

    pltpu.sync_copy(work.at[_RES], out_hbm.at[pl.ds(wid * _OUT, _OUT)])


@jax.jit
def _lamaface_sc(label, fn):
    mesh = plsc.VectorSubcoreMesh(core_axis_name="c", subcore_axis_name="s")
    run = pl.kernel(
        _sc_body,
        out_type=jax.ShapeDtypeStruct((_BATCH,), jnp.float32),
        mesh=mesh,
        scratch_types=[
            pltpu.VMEM((2, _HALF), jnp.int32),
            pltpu.VMEM((2, _HALF), jnp.float32),
            pltpu.VMEM((2, _HALF), jnp.float32),
            pltpu.VMEM((6, _HALF), jnp.float32),
            pltpu.VMEM_SHARED((_C_PAD,), jnp.float32),
            pltpu.VMEM_SHARED((_C_PAD,), jnp.float32),
            pltpu.VMEM_SHARED((_C_PAD,), jnp.float32),
            pltpu.SemaphoreType.DMA,
            pltpu.SemaphoreType.DMA,
            pltpu.SemaphoreType.DMA,
        ],
    )
    return run(label, fn)


def kernel(feature_norm, label, kernel):
    del kernel
    res = _lamaface_sc(label, feature_norm[:, 0])
    return res[:, None]


# speedup vs baseline: 9.0242x; 1.0248x over previous
import jax
import jax.numpy as jnp
from jax import lax
from jax.experimental import pallas as pl
from jax.experimental.pallas import tpu as pltpu, tpu_sc as plsc

_CLASSNUM = 70722
_BATCH = 4096
_EPS = 0.001

_NC = 2
_NS = 16
_L = 16
_C_PAD = 70728
_CHUNK = _BATCH // _NS
_HALF = _CHUNK // 2
_OUT = _BATCH // (_NC * _NS)

_FN = 0
_SQ2 = 2
_ONES, _ZEROS, _CNT, _SUM, _SQ, _RES = 4, 5, 6, 7, 8, 9


def _newton_rsqrt(v):
    i = lax.bitcast_convert_type(v, jnp.int32)
    i = jnp.int32(0x5F3759DF) - lax.shift_right_logical(i, 1)
    y = lax.bitcast_convert_type(i, jnp.float32)
    for _ in range(3):
        y = y * (1.5 - 0.5 * v * y * y)
    return y


def _sc_body(label_hbm, fn_hbm, out_hbm,
             lab2, buf, sem_a, sem_b, sem_c,
             counts_sh, sums_sh, sqs_sh):
    cid = lax.axis_index("c")
    sid = lax.axis_index("s")
    wid = sid * _NC + cid

    lab_d = []
    fn_d = []
    for j in range(2):
        base = sid * _CHUNK + j * _HALF
        lab_d.append(pltpu.async_copy(
            label_hbm.at[pl.ds(base, _HALF)], lab2.at[j], sem_a))
        fn_d.append(pltpu.async_copy(
            fn_hbm.at[pl.ds(base, _HALF)], buf.at[_FN + j], sem_b))

    def fill(k, _):
        sl = pl.ds(k * _L, _L)
        buf[_ONES, sl] = jnp.full((_L,), 1.0, jnp.float32)
        buf[_ZEROS, sl] = jnp.full((_L,), 0.0, jnp.float32)
        return 0
    lax.fori_loop(0, _HALF // _L, fill, 0)

    for d in lab_d:
        d.wait()

    zero_d = []
    for j in range(2):
        idx = lab2.at[j]
        zero_d.append(pltpu.async_copy(buf.at[_ZEROS], counts_sh.at[idx], sem_c))
        zero_d.append(pltpu.async_copy(buf.at[_ZEROS], sums_sh.at[idx], sem_c))
        zero_d.append(pltpu.async_copy(buf.at[_ZEROS], sqs_sh.at[idx], sem_c))

    for d in fn_d:
        d.wait()

    def square(k, _):
        sl = pl.ds(k * _L, _L)
        for j in range(2):
            f = buf[_FN + j, sl]
            buf[_SQ2 + j, sl] = f * f
        return 0
    lax.fori_loop(0, _HALF // _L, square, 0)

    for d in zero_d:
        d.wait()
    plsc.subcore_barrier()

    add_d = []
    for j in range(2):
        idx = lab2.at[j]
        add_d.append(pltpu.async_copy(
            buf.at[_ONES], counts_sh.at[idx], sem_a, add=True))
        add_d.append(pltpu.async_copy(
            buf.at[_FN + j], sums_sh.at[idx], sem_a, add=True))
        add_d.append(pltpu.async_copy(
            buf.at[_SQ2 + j], sqs_sh.at[idx], sem_a, add=True))
    for d in add_d:
        d.wait()
    plsc.subcore_barrier()

    idx = lab2.at[cid]
    gat_d = [
        pltpu.async_copy(counts_sh.at[idx], buf.at[_CNT], sem_b),
        pltpu.async_copy(sums_sh.at[idx], buf.at[_SUM], sem_b),
        pltpu.async_copy(sqs_sh.at[idx], buf.at[_SQ], sem_b),
    ]
    for d in gat_d:
        d.wait()

    def norm(k, _):
        sl = pl.ds(k * _L, _L)
        cnt = buf[_CNT, sl]
        s = buf[_SUM, sl]
        q = buf[_SQ, sl]
        f = buf[_FN + cid, sl]
        mean = s / jnp.maximum(cnt, 1.0)
        var = (q - cnt * mean * mean) / jnp.maximum(cnt - 1.0, 1.0)
        var = jnp.maximum(var, 0.0)
        y = _newton_rsqrt(jnp.maximum(var, 1e-30))
        std = var * y
        d = f - mean
        buf[_RES, sl] = jnp.where(cnt > 2.0, d / (std + _EPS), d / 20.0)
        return 0
    lax.fori_loop(0, _OUT // _L, norm, 0)

    pltpu.sync_copy(buf.at[_RES], out_hbm.at[pl.ds(wid * _OUT, _OUT)])


@jax.jit
def _lamaface_sc(label, fn):
    mesh = plsc.VectorSubcoreMesh(core_axis_name="c", subcore_axis_name="s")
    run = pl.kernel(
        _sc_body,
        out_type=jax.ShapeDtypeStruct((_BATCH,), jnp.float32),
        mesh=mesh,
        scratch_types=[
            pltpu.VMEM((2, _HALF), jnp.int32),
            pltpu.VMEM((10, _HALF), jnp.float32),
            pltpu.SemaphoreType.DMA,
            pltpu.SemaphoreType.DMA,
            pltpu.SemaphoreType.DMA,
            pltpu.VMEM_SHARED((_C_PAD,), jnp.float32),
            pltpu.VMEM_SHARED((_C_PAD,), jnp.float32),
            pltpu.VMEM_SHARED((_C_PAD,), jnp.float32),
        ],
    )
    return run(label, fn)


def kernel(feature_norm, label, kernel):
    del kernel
    res = _lamaface_sc(label, feature_norm[:, 0])
    return res[:, None]


# speedup vs baseline: 9.2618x; 1.0263x over previous
import jax
import jax.numpy as jnp
from jax import lax
from jax.experimental import pallas as pl
from jax.experimental.pallas import tpu as pltpu, tpu_sc as plsc

_CLASSNUM = 70722
_BATCH = 4096
_EPS = 0.001

_NS = 16
_L = 16
_C_PAD = 70728
_CHUNK = _BATCH // _NS
_HALF = _CHUNK // 2

_FN = 0
_SQ2 = 2
_ONES, _ZEROS = 4, 5
_CNT, _SUM, _SQ, _RES = 6, 8, 10, 12


def _newton_rsqrt(v):
    i = lax.bitcast_convert_type(v, jnp.int32)
    i = jnp.int32(0x5F3759DF) - lax.shift_right_logical(i, 1)
    y = lax.bitcast_convert_type(i, jnp.float32)
    for _ in range(3):
        y = y * (1.5 - 0.5 * v * y * y)
    return y


def _sc_body(label_hbm, fn_hbm, out_hbm,
             idx6, buf, sem_a, sem_b, sem_c, tbl_sh):
    sid = lax.axis_index("s")

    lab_d = []
    fn_d = []
    for j in range(2):
        base = sid * _CHUNK + j * _HALF
        lab_d.append(pltpu.async_copy(
            label_hbm.at[pl.ds(base, _HALF)], idx6.at[j], sem_a))
        fn_d.append(pltpu.async_copy(
            fn_hbm.at[pl.ds(base, _HALF)], buf.at[_FN + j], sem_b))

    def fill(k, _):
        sl = pl.ds(k * _L, _L)
        buf[_ONES, sl] = jnp.full((_L,), 1.0, jnp.float32)
        buf[_ZEROS, sl] = jnp.full((_L,), 0.0, jnp.float32)
        return 0
    lax.fori_loop(0, _HALF // _L, fill, 0)

    for d in lab_d:
        d.wait()

    def offs(k, _):
        sl = pl.ds(k * _L, _L)
        for j in range(2):
            lab = idx6[j, sl]
            idx6[2 + j, sl] = lab + _C_PAD
            idx6[4 + j, sl] = lab + 2 * _C_PAD
        return 0
    lax.fori_loop(0, _HALF // _L, offs, 0)

    zero_d = []
    for r in range(6):
        zero_d.append(pltpu.async_copy(
            buf.at[_ZEROS], tbl_sh.at[idx6.at[r]], sem_c))

    for d in fn_d:
        d.wait()

    def square(k, _):
        sl = pl.ds(k * _L, _L)
        for j in range(2):
            f = buf[_FN + j, sl]
            buf[_SQ2 + j, sl] = f * f
        return 0
    lax.fori_loop(0, _HALF // _L, square, 0)

    for d in zero_d:
        d.wait()
    plsc.subcore_barrier()

    add_src = [_ONES, _ONES, _FN, _FN + 1, _SQ2, _SQ2 + 1]
    add_d = []
    for r in range(6):
        add_d.append(pltpu.async_copy(
            buf.at[add_src[r]], tbl_sh.at[idx6.at[r]], sem_a, add=True))
    for d in add_d:
        d.wait()
    plsc.subcore_barrier()

    gat_dst = [_CNT, _CNT + 1, _SUM, _SUM + 1, _SQ, _SQ + 1]
    gat_d = []
    for r in range(6):
        gat_d.append(pltpu.async_copy(
            tbl_sh.at[idx6.at[r]], buf.at[gat_dst[r]], sem_b))
    for d in gat_d:
        d.wait()

    def norm(k, _):
        sl = pl.ds(k * _L, _L)
        for j in range(2):
            cnt = buf[_CNT + j, sl]
            s = buf[_SUM + j, sl]
            q = buf[_SQ + j, sl]
            f = buf[_FN + j, sl]
            mean = s / jnp.maximum(cnt, 1.0)
            var = (q - cnt * mean * mean) / jnp.maximum(cnt - 1.0, 1.0)
            var = jnp.maximum(var, 0.0)
            y = _newton_rsqrt(jnp.maximum(var, 1e-30))
            std = var * y
            d = f - mean
            buf[_RES + j, sl] = jnp.where(cnt > 2.0, d / (std + _EPS), d / 20.0)
        return 0
    lax.fori_loop(0, _HALF // _L, norm, 0)

    for j in range(2):
        pltpu.sync_copy(buf.at[_RES + j],
                        out_hbm.at[pl.ds(sid * _CHUNK + j * _HALF, _HALF)])


@jax.jit
def _lamaface_sc(label, fn):
    mesh = plsc.VectorSubcoreMesh(core_axis_name="c", subcore_axis_name="s",
                                  num_cores=1)
    run = pl.kernel(
        _sc_body,
        out_type=jax.ShapeDtypeStruct((_BATCH,), jnp.float32),
        mesh=mesh,
        scratch_types=[
            pltpu.VMEM((6, _HALF), jnp.int32),
            pltpu.VMEM((14, _HALF), jnp.float32),
            pltpu.SemaphoreType.DMA,
            pltpu.SemaphoreType.DMA,
            pltpu.SemaphoreType.DMA,
            pltpu.VMEM_SHARED((3 * _C_PAD,), jnp.float32),
        ],
    )
    return run(label, fn)


def kernel(feature_norm, label, kernel):
    del kernel
    res = _lamaface_sc(label, feature_norm[:, 0])
    return res[:, None]


# speedup vs baseline: 9.3935x; 1.0142x over previous
import jax
import jax.numpy as jnp
from jax import lax
from jax.experimental import pallas as pl
from jax.experimental.pallas import tpu as pltpu, tpu_sc as plsc

_CLASSNUM = 70722
_BATCH = 4096
_EPS = 0.001

_NC = 1
_NS = 16
_L = 16
_C_PAD = 70728
_CHUNK = _BATCH // _NS
_HALF = _CHUNK // 2
_OUT = _BATCH // (_NC * _NS)

_FN = 0
_SQ2 = 2
_ONES, _ZEROS = 4, 5
_CNT, _SUM, _SQ, _RES = 6, 8, 10, 12


def _newton_rsqrt(v):
    i = lax.bitcast_convert_type(v, jnp.int32)
    i = jnp.int32(0x5F3759DF) - lax.shift_right_logical(i, 1)
    y = lax.bitcast_convert_type(i, jnp.float32)
    for _ in range(3):
        y = y * (1.5 - 0.5 * v * y * y)
    return y


def _sc_body(label_hbm, fn_hbm, out_hbm,
             lab2, buf, sem_a, sem_b, sem_c,
             counts_sh, sums_sh, sqs_sh):
    sid = lax.axis_index("s")

    lab_d = []
    fn_d = []
    for j in range(2):
        base = sid * _CHUNK + j * _HALF
        lab_d.append(pltpu.async_copy(
            label_hbm.at[pl.ds(base, _HALF)], lab2.at[j], sem_a))
        fn_d.append(pltpu.async_copy(
            fn_hbm.at[pl.ds(base, _HALF)], buf.at[_FN + j], sem_b))

    def fill(k, _):
        sl = pl.ds(k * _L, _L)
        buf[_ONES, sl] = jnp.full((_L,), 1.0, jnp.float32)
        buf[_ZEROS, sl] = jnp.full((_L,), 0.0, jnp.float32)
        return 0
    lax.fori_loop(0, _HALF // _L, fill, 0)

    for d in lab_d:
        d.wait()

    zero_d = []
    for j in range(2):
        idx = lab2.at[j]
        zero_d.append(pltpu.async_copy(buf.at[_ZEROS], counts_sh.at[idx], sem_c))
        zero_d.append(pltpu.async_copy(buf.at[_ZEROS], sums_sh.at[idx], sem_c))
        zero_d.append(pltpu.async_copy(buf.at[_ZEROS], sqs_sh.at[idx], sem_c))

    for d in fn_d:
        d.wait()

    def square(k, _):
        sl = pl.ds(k * _L, _L)
        for j in range(2):
            f = buf[_FN + j, sl]
            buf[_SQ2 + j, sl] = f * f
        return 0
    lax.fori_loop(0, _HALF // _L, square, 0)

    for d in zero_d:
        d.wait()
    plsc.subcore_barrier()

    add_d = []
    for j in range(2):
        idx = lab2.at[j]
        add_d.append(pltpu.async_copy(
            buf.at[_ONES], counts_sh.at[idx], sem_a, add=True))
        add_d.append(pltpu.async_copy(
            buf.at[_FN + j], sums_sh.at[idx], sem_a, add=True))
        add_d.append(pltpu.async_copy(
            buf.at[_SQ2 + j], sqs_sh.at[idx], sem_a, add=True))
    for d in add_d:
        d.wait()
    plsc.subcore_barrier()

    gat_d = [[], []]
    for j, sem in ((0, sem_b), (1, sem_c)):
        idx = lab2.at[j]
        gat_d[j].append(pltpu.async_copy(counts_sh.at[idx], buf.at[_CNT + j], sem))
        gat_d[j].append(pltpu.async_copy(sums_sh.at[idx], buf.at[_SUM + j], sem))
        gat_d[j].append(pltpu.async_copy(sqs_sh.at[idx], buf.at[_SQ + j], sem))

    def make_norm(j):
        def norm(k, _):
            sl = pl.ds(k * _L, _L)
            cnt = buf[_CNT + j, sl]
            s = buf[_SUM + j, sl]
            q = buf[_SQ + j, sl]
            f = buf[_FN + j, sl]
            mean = s / jnp.maximum(cnt, 1.0)
            var = (q - cnt * mean * mean) / jnp.maximum(cnt - 1.0, 1.0)
            var = jnp.maximum(var, 0.0)
            y = _newton_rsqrt(jnp.maximum(var, 1e-30))
            denom = jnp.where(cnt > 2.0, var * y + _EPS, 20.0)
            buf[_RES + j, sl] = (f - mean) / denom
            return 0
        return norm

    for j in range(2):
        for d in gat_d[j]:
            d.wait()
        lax.fori_loop(0, _HALF // _L, make_norm(j), 0)
        pltpu.sync_copy(buf.at[_RES + j],
                        out_hbm.at[pl.ds(sid * _CHUNK + j * _HALF, _HALF)])


@jax.jit
def _lamaface_sc(label, fn):
    mesh = plsc.VectorSubcoreMesh(core_axis_name="c", subcore_axis_name="s", num_cores=1)
    run = pl.kernel(
        _sc_body,
        out_type=jax.ShapeDtypeStruct((_BATCH,), jnp.float32),
        mesh=mesh,
        scratch_types=[
            pltpu.VMEM((2, _HALF), jnp.int32),
            pltpu.VMEM((14, _HALF), jnp.float32),
            pltpu.SemaphoreType.DMA,
            pltpu.SemaphoreType.DMA,
            pltpu.SemaphoreType.DMA,
            pltpu.VMEM_SHARED((_C_PAD,), jnp.float32),
            pltpu.VMEM_SHARED((_C_PAD,), jnp.float32),
            pltpu.VMEM_SHARED((_C_PAD,), jnp.float32),
        ],
    )
    return run(label, fn)


def kernel(feature_norm, label, kernel):
    del kernel
    res = _lamaface_sc(label, feature_norm[:, 0])
    return res[:, None]


# speedup vs baseline: 9.4984x; 1.0112x over previous
import jax
import jax.numpy as jnp
from jax import lax
from jax.experimental import pallas as pl
from jax.experimental.pallas import tpu as pltpu, tpu_sc as plsc

_CLASSNUM = 70722
_BATCH = 4096
_EPS = 0.001

_NC = 1
_NS = 16
_L = 16
_C_PAD = 70728
_CHUNK = _BATCH // _NS
_HALF = _CHUNK // 2
_OUT = _BATCH // (_NC * _NS)

_FN = 0
_SQ2 = 2
_ONES, _ZEROS = 4, 5
_CNT, _SUM, _SQ, _RES = 6, 8, 10, 12


def _newton_rsqrt(v):
    i = lax.bitcast_convert_type(v, jnp.int32)
    i = jnp.int32(0x5F3759DF) - lax.shift_right_logical(i, 1)
    y = lax.bitcast_convert_type(i, jnp.float32)
    for _ in range(3):
        y = y * (1.5 - 0.5 * v * y * y)
    return y


def _sc_body(label_hbm, fn_hbm, out_hbm,
             lab2, buf, sem_a, sem_b, sem_c,
             counts_sh, sums_sh, sqs_sh):
    sid = lax.axis_index("s")

    lab_d = []
    fn_d = []
    for j in range(2):
        base = sid * _CHUNK + j * _HALF
        lab_d.append(pltpu.async_copy(
            label_hbm.at[pl.ds(base, _HALF)], lab2.at[j], sem_a))
        fn_d.append(pltpu.async_copy(
            fn_hbm.at[pl.ds(base, _HALF)], buf.at[_FN + j], sem_b))

    def fill(k, _):
        sl = pl.ds(k * _L, _L)
        buf[_ONES, sl] = jnp.full((_L,), 1.0, jnp.float32)
        buf[_ZEROS, sl] = jnp.full((_L,), 0.0, jnp.float32)
        return 0
    lax.fori_loop(0, _HALF // _L, fill, 0)

    for d in lab_d:
        d.wait()

    zero_d = []
    for j in range(2):
        idx = lab2.at[j]
        zero_d.append(pltpu.async_copy(buf.at[_ZEROS], counts_sh.at[idx], sem_c))
        zero_d.append(pltpu.async_copy(buf.at[_ZEROS], sums_sh.at[idx], sem_c))
        zero_d.append(pltpu.async_copy(buf.at[_ZEROS], sqs_sh.at[idx], sem_c))

    for d in fn_d:
        d.wait()

    def square(k, _):
        sl = pl.ds(k * _L, _L)
        for j in range(2):
            f = buf[_FN + j, sl]
            buf[_SQ2 + j, sl] = f * f
        return 0
    lax.fori_loop(0, _HALF // _L, square, 0)

    for d in zero_d:
        d.wait()
    plsc.subcore_barrier()

    add_d = []
    for j in range(2):
        idx = lab2.at[j]
        add_d.append(pltpu.async_copy(
            buf.at[_ONES], counts_sh.at[idx], sem_a, add=True))
        add_d.append(pltpu.async_copy(
            buf.at[_FN + j], sums_sh.at[idx], sem_a, add=True))
        add_d.append(pltpu.async_copy(
            buf.at[_SQ2 + j], sqs_sh.at[idx], sem_a, add=True))
    for d in add_d:
        d.wait()
    plsc.subcore_barrier()

    gat_d = []
    for j in range(2):
        idx = lab2.at[j]
        gat_d.append(pltpu.async_copy(counts_sh.at[idx], buf.at[_CNT + j], sem_b))
        gat_d.append(pltpu.async_copy(sums_sh.at[idx], buf.at[_SUM + j], sem_b))
        gat_d.append(pltpu.async_copy(sqs_sh.at[idx], buf.at[_SQ + j], sem_b))
    for d in gat_d:
        d.wait()

    def norm(k, _):
        sl = pl.ds(k * _L, _L)
        for j in range(2):
            cnt = buf[_CNT + j, sl]
            s = buf[_SUM + j, sl]
            q = buf[_SQ + j, sl]
            f = buf[_FN + j, sl]
            mean = s / jnp.maximum(cnt, 1.0)
            var = (q - cnt * mean * mean) / jnp.maximum(cnt - 1.0, 1.0)
            var = jnp.maximum(var, 0.0)
            y = _newton_rsqrt(jnp.maximum(var, 1e-30))
            std = var * y
            d = f - mean
            buf[_RES + j, sl] = jnp.where(cnt > 2.0, d / (std + _EPS), d / 20.0)
        return 0
    lax.fori_loop(0, _HALF // _L, norm, 0)

    for j in range(2):
        pltpu.sync_copy(buf.at[_RES + j],
                        out_hbm.at[pl.ds(sid * _CHUNK + j * _HALF, _HALF)])


@jax.jit
def _lamaface_sc(label, fn):
    mesh = plsc.VectorSubcoreMesh(core_axis_name="c", subcore_axis_name="s", num_cores=1)
    run = pl.kernel(
        _sc_body,
        out_type=jax.ShapeDtypeStruct((_BATCH,), jnp.float32),
        mesh=mesh,
        scratch_types=[
            pltpu.VMEM((2, _HALF), jnp.int32),
            pltpu.VMEM((14, _HALF), jnp.float32),
            pltpu.SemaphoreType.DMA,
            pltpu.SemaphoreType.DMA,
            pltpu.SemaphoreType.DMA,
            pltpu.VMEM_SHARED((_C_PAD,), jnp.float32),
            pltpu.VMEM_SHARED((_C_PAD,), jnp.float32),
            pltpu.VMEM_SHARED((_C_PAD,), jnp.float32),
        ],
    )
    return run(label, fn)


def kernel(feature_norm, label, kernel):
    del kernel
    res = _lamaface_sc(label, feature_norm[:, 0])
    return res[:, None]
